# baseline (device time: 270169 ns/iter reference)
import jax
import jax.numpy as jnp
from jax import lax
from jax.experimental import pallas as pl
from jax.experimental.pallas import tpu as pltpu

N_DEV = 8

X, Y, Z = 1, 3, 4

THIRDS = (
    (0, 1376, (X, Y, Z)),
    (1376, 1360, (Y, Z, X)),
    (2736, 1360, (Z, X, Y)),
)


def kernel(x):
    m_per, n = x.shape
    assert m_per == 4096

    def body(x_ref, out_ref, stage_ref, local_sem, send_sems, recv_sems):
        my = lax.axis_index("i")

        barrier_sem = pltpu.get_barrier_semaphore()
        for mask in (X, Y, Z):
            pl.semaphore_signal(
                barrier_sem, inc=1,
                device_id=(my ^ mask,), device_id_type=pl.DeviceIdType.MESH,
            )
        pl.semaphore_wait(barrier_sem, 3)

        stage_ref[...] = x_ref[...].astype(jnp.bfloat16)
        cp = pltpu.make_async_copy(
            stage_ref, out_ref.at[pl.ds(my * m_per, m_per)], local_sem
        )
        cp.start()

        def held_origins(dims, k):
            acc = [0]
            for e in dims[:k]:
                acc = acc + [o ^ e for o in acc]
            return acc

        def phase_sends(t, k):
            r0, mt, dims = THIRDS[t]
            base = (1 << k) - 1
            partner = my ^ dims[k]
            descs = []
            for j, rel in enumerate(held_origins(dims, k)):
                o = my ^ rel
                dst = out_ref.at[pl.ds(o * m_per + r0, mt)]
                src = (stage_ref.at[pl.ds(r0, mt)] if rel == 0
                       else out_ref.at[pl.ds(o * m_per + r0, mt)])
                d = pltpu.make_async_remote_copy(
                    src_ref=src, dst_ref=dst,
                    send_sem=send_sems.at[t, base + j],
                    recv_sem=recv_sems.at[t, base + j],
                    device_id=(partner,),
                    device_id_type=pl.DeviceIdType.MESH,
                )
                d.start()
                descs.append(d)
            return descs

        def phase_recv_wait(t, k):
            r0, mt, dims = THIRDS[t]
            base = (1 << k) - 1
            partner = my ^ dims[k]
            for j, rel in enumerate(held_origins(dims, k)):
                o = partner ^ rel
                dst = out_ref.at[pl.ds(o * m_per + r0, mt)]
                pltpu.make_async_remote_copy(
                    src_ref=dst, dst_ref=dst,
                    send_sem=send_sems.at[t, base + j],
                    recv_sem=recv_sems.at[t, base + j],
                    device_id=(partner,),
                    device_id_type=pl.DeviceIdType.MESH,
                ).wait_recv()

        pending = []
        for t in range(3):
            pending += phase_sends(t, 0)
        for k in range(1, 3):
            for t in range(3):
                phase_recv_wait(t, k - 1)
                pending += phase_sends(t, k)
        for t in range(3):
            phase_recv_wait(t, 2)
        for d in pending:
            d.wait_send()
        cp.wait()

    return pl.pallas_call(
        body,
        out_shape=jax.ShapeDtypeStruct((N_DEV * m_per, n), jnp.bfloat16),
        in_specs=[pl.BlockSpec(memory_space=pltpu.VMEM)],
        out_specs=pl.BlockSpec(memory_space=pl.ANY),
        scratch_shapes=[
            pltpu.VMEM((m_per, n), jnp.bfloat16),
            pltpu.SemaphoreType.DMA,
            pltpu.SemaphoreType.DMA((3, 7)),
            pltpu.SemaphoreType.DMA((3, 7)),
        ],
        compiler_params=pltpu.CompilerParams(collective_id=0),
    )(x)
